# baseline (speedup 1.0000x reference)
import functools
import jax
import jax.numpy as jnp
from jax import lax
from jax.experimental import pallas as pl
from jax.experimental.pallas import tpu as pltpu
from jax.experimental.pallas import tpu_sc as plsc

PCR = (0.0, -39.68, -3.0, 69.12, 39.68, 1.0)
BEV_SIZE = 0.32
BEV_W = int(round((PCR[3] - PCR[0]) / BEV_SIZE))
BEV_H = int(round((PCR[4] - PCR[1]) / BEV_SIZE))

P = 1024
NPAD = 409600
NW = 16
SC_CHUNK = 128
ZROWS = 672
ZREP = 20
CROWS = NW * ZREP * ZROWS
EPS = 1e-3


def _stats1_kernel(x_ref, s_ref, xx_ref):
    @pl.when(pl.program_id(0) == 0)
    def _():
        s_ref[...] = jnp.zeros_like(s_ref)
        xx_ref[...] = jnp.zeros_like(xx_ref)
    x = x_ref[...]
    s_ref[...] += jnp.sum(x, axis=0, keepdims=True)
    xx_ref[...] += lax.dot_general(x, x, (((0,), (0,)), ((), ())),
                                   preferred_element_type=jnp.float32)


def _stats2_kernel(n_real, x_ref, a1_ref, b1_ref, s_ref, hh_ref):
    @pl.when(pl.program_id(0) == 0)
    def _():
        s_ref[...] = jnp.zeros_like(s_ref)
        hh_ref[...] = jnp.zeros_like(hh_ref)
    i = pl.program_id(0)
    h1 = jax.nn.relu(jnp.dot(x_ref[...], a1_ref[...],
                             preferred_element_type=jnp.float32) + b1_ref[...])
    rid = i * P + lax.broadcasted_iota(jnp.int32, (P, 1), 0)
    h1 = jnp.where(rid < n_real, h1, 0.0)
    s_ref[...] += jnp.sum(h1, axis=0, keepdims=True)
    hh_ref[...] += lax.dot_general(h1, h1, (((0,), (0,)), ((), ())),
                                   preferred_element_type=jnp.float32)


def _shift(a, d, fill):
    pad = jnp.full((d, a.shape[1]), fill, a.dtype)
    return jnp.concatenate([pad, a[:-d]], axis=0)


def _fwd_scan_kernel(dump, x_ref, k_ref, a1_ref, b1_ref, a2_ref, b2_ref,
                     ws_ref, bs_ref, val_ref, idx_ref,
                     ck_ref, ct_ref, ce_ref, cm_ref):
    @pl.when(pl.program_id(0) == 0)
    def _():
        ck_ref[...] = jnp.full((1, 1), -1, jnp.int32)
        ct_ref[...] = jnp.zeros_like(ct_ref)
        ce_ref[...] = jnp.zeros_like(ce_ref)
        cm_ref[...] = jnp.zeros_like(cm_ref)

    x = x_ref[...]
    k = k_ref[...]
    h1 = jax.nn.relu(jnp.dot(x, a1_ref[...],
                             preferred_element_type=jnp.float32) + b1_ref[...])
    h2 = jax.nn.relu(jnp.dot(h1, a2_ref[...],
                             preferred_element_type=jnp.float32) + b2_ref[...])
    score = jax.nn.relu(jnp.dot(h2, ws_ref[...],
                                preferred_element_type=jnp.float32) + bs_ref[...])
    e = jnp.exp(score)
    t = h2 * e

    kprev = jnp.concatenate([ck_ref[...], k[:-1]], axis=0)
    same = (k == kprev)
    row0 = lax.broadcasted_iota(jnp.int32, (P, 1), 0) == 0
    seed = row0 & same
    s_t = t + jnp.where(seed, ct_ref[...], 0.0)
    s_e = e + jnp.where(seed, ce_ref[...], 0.0)
    s_m = jnp.where(seed, jnp.maximum(h2, cm_ref[...]), h2)
    g = same.astype(jnp.float32)

    d = 1
    while d < P:
        gs = _shift(g, d, 0.0)
        s_t = s_t + g * _shift(s_t, d, 0.0)
        s_e = s_e + g * _shift(s_e, d, 0.0)
        s_m = jnp.maximum(s_m, jnp.where(g > 0, _shift(s_m, d, 0.0), 0.0))
        g = g * gs
        d *= 2

    tp = jnp.concatenate([ct_ref[...], s_t[:-1]], axis=0)
    ep = jnp.concatenate([ce_ref[...], s_e[:-1]], axis=0)
    mp = jnp.concatenate([cm_ref[...], s_m[:-1]], axis=0)
    emit = jnp.logical_not(same) & (kprev >= 0)
    val = (tp / ep + mp) * 0.5
    val = jnp.concatenate([val, jnp.zeros((P, 64), jnp.float32)], axis=1)
    val_ref[...] = jnp.where(emit, val, 0.0)
    idx_ref[...] = jnp.where(emit, kprev, dump)

    ck_ref[...] = k[P - 1:P]
    ct_ref[...] = s_t[P - 1:P]
    ce_ref[...] = s_e[P - 1:P]
    cm_ref[...] = s_m[P - 1:P]


def _make_sc_scatter():
    mesh = plsc.VectorSubcoreMesh(core_axis_name="c", subcore_axis_name="s",
                                  num_cores=1)
    pts_per_w = NPAD // NW
    n_iter = pts_per_w // SC_CHUNK

    @functools.partial(
        pl.kernel, mesh=mesh,
        out_type=jax.ShapeDtypeStruct((CROWS, 128), jnp.float32),
        scratch_types=[
            pltpu.VMEM((ZROWS, 128), jnp.float32),
            pltpu.VMEM((SC_CHUNK,), jnp.int32),
            pltpu.VMEM((SC_CHUNK, 128), jnp.float32),
        ],
    )
    def scatter(vals_hbm, idx_hbm, z_hbm, out_hbm, zbuf, idx_v, rows_v):
        w = lax.axis_index("s")
        pltpu.sync_copy(z_hbm, zbuf)
        zbase = w * (ZREP * ZROWS)
        for j in range(ZREP):
            pltpu.sync_copy(zbuf, out_hbm.at[pl.ds(zbase + j * ZROWS, ZROWS)])
        plsc.subcore_barrier()
        pbase = w * pts_per_w

        def step(i, carry):
            off = pbase + i * SC_CHUNK
            pltpu.sync_copy(idx_hbm.at[pl.ds(off, SC_CHUNK)], idx_v)
            pltpu.sync_copy(vals_hbm.at[pl.ds(off, SC_CHUNK)], rows_v)
            pltpu.sync_copy(rows_v, out_hbm.at[idx_v])
            return carry

        lax.fori_loop(0, n_iter, step, 0)

    return scatter


_sc_scatter = _make_sc_scatter()


def kernel(xyz, xyz_batch_cnt, pt_feature, W1, W2, Ws, bs):
    nb = int(xyz_batch_cnt.shape[0])
    n = int(xyz.shape[0])
    m = nb * BEV_W * BEV_H
    dump = m

    batch_id = jnp.repeat(jnp.arange(nb, dtype=jnp.int32), xyz_batch_cnt,
                          total_repeat_length=n)
    xi = jnp.clip(jnp.floor((xyz[:, 0] - PCR[0]) / BEV_SIZE), 0, BEV_W - 1).astype(jnp.int32)
    yi = jnp.clip(jnp.floor((xyz[:, 1] - PCR[1]) / BEV_SIZE), 0, BEV_H - 1).astype(jnp.int32)
    keys = batch_id * (BEV_W * BEV_H) + xi * BEV_H + yi
    cx = (xi.astype(jnp.float32) + 0.5) * BEV_SIZE + PCR[0]
    cy = (yi.astype(jnp.float32) + 0.5) * BEV_SIZE + PCR[1]
    cz = jnp.full_like(cx, 0.5 * (PCR[2] + PCR[5]))
    gf = jnp.concatenate([pt_feature, xyz - jnp.stack([cx, cy, cz], axis=1)],
                         axis=1)

    order = jnp.argsort(keys)
    ks = jnp.concatenate([keys[order],
                          jnp.full((NPAD - n,), dump, jnp.int32)])[:, None]
    xs = jnp.concatenate([gf[order], jnp.zeros((NPAD - n, 8), jnp.float32)])

    grid = NPAD // P
    sum_x, xx = pl.pallas_call(
        _stats1_kernel,
        grid=(grid,),
        in_specs=[pl.BlockSpec((P, 8), lambda i: (i, 0))],
        out_specs=[pl.BlockSpec((1, 8), lambda i: (0, 0)),
                   pl.BlockSpec((8, 8), lambda i: (0, 0))],
        out_shape=[jax.ShapeDtypeStruct((1, 8), jnp.float32),
                   jax.ShapeDtypeStruct((8, 8), jnp.float32)],
    )(xs)

    m1 = (sum_x / n) @ W1
    var1 = jnp.diagonal(W1.T @ (xx / n) @ W1)[None] - m1 * m1
    inv1 = lax.rsqrt(var1 + EPS)
    A1 = W1 * inv1
    b1 = -m1 * inv1

    sum_h, hh = pl.pallas_call(
        functools.partial(_stats2_kernel, n),
        grid=(grid,),
        in_specs=[pl.BlockSpec((P, 8), lambda i: (i, 0)),
                  pl.BlockSpec((8, 32), lambda i: (0, 0)),
                  pl.BlockSpec((1, 32), lambda i: (0, 0))],
        out_specs=[pl.BlockSpec((1, 32), lambda i: (0, 0)),
                   pl.BlockSpec((32, 32), lambda i: (0, 0))],
        out_shape=[jax.ShapeDtypeStruct((1, 32), jnp.float32),
                   jax.ShapeDtypeStruct((32, 32), jnp.float32)],
    )(xs, A1, b1)

    m2 = (sum_h / n) @ W2
    var2 = jnp.diagonal(W2.T @ (hh / n) @ W2)[None] - m2 * m2
    inv2 = lax.rsqrt(var2 + EPS)
    A2 = W2 * inv2
    b2 = -m2 * inv2

    vals, idx = pl.pallas_call(
        functools.partial(_fwd_scan_kernel, dump),
        grid=(grid,),
        in_specs=[pl.BlockSpec((P, 8), lambda i: (i, 0)),
                  pl.BlockSpec((P, 1), lambda i: (i, 0)),
                  pl.BlockSpec((8, 32), lambda i: (0, 0)),
                  pl.BlockSpec((1, 32), lambda i: (0, 0)),
                  pl.BlockSpec((32, 64), lambda i: (0, 0)),
                  pl.BlockSpec((1, 64), lambda i: (0, 0)),
                  pl.BlockSpec((64, 64), lambda i: (0, 0)),
                  pl.BlockSpec((1, 64), lambda i: (0, 0))],
        out_specs=[pl.BlockSpec((P, 128), lambda i: (i, 0)),
                   pl.BlockSpec((P, 1), lambda i: (i, 0))],
        out_shape=[jax.ShapeDtypeStruct((NPAD, 128), jnp.float32),
                   jax.ShapeDtypeStruct((NPAD, 1), jnp.int32)],
        scratch_shapes=[pltpu.VMEM((1, 1), jnp.int32),
                        pltpu.VMEM((1, 64), jnp.float32),
                        pltpu.VMEM((1, 64), jnp.float32),
                        pltpu.VMEM((1, 64), jnp.float32)],
    )(xs, ks, A1, b1, A2, b2, Ws, bs[None, :])

    canvas = _sc_scatter(vals, idx[:, 0],
                         jnp.zeros((ZROWS, 128), jnp.float32))
    return (canvas[:m, :64].reshape(nb, BEV_W * BEV_H, 64)
            .transpose(0, 2, 1).reshape(nb, 64, BEV_W, BEV_H))
